# Initial kernel scaffold; baseline (speedup 1.0000x reference)
import jax
import jax.numpy as jnp
from jax.experimental import pallas as pl


def kernel(token_ids, lookup_table):
    raise NotImplementedError("write your pallas kernel here")


# speedup vs baseline: 1.6824x; 1.6824x over previous
import functools

import jax
import jax.numpy as jnp
from jax import lax
from jax.experimental import pallas as pl
from jax.experimental.pallas import tpu as pltpu
from jax.experimental.pallas import tpu_sc as plsc

NUM_EMBEDDINGS = 1000000
EMBED_DIM = 64
BATCH = 16384
HIST = 50

NC = 2
NS = 16
NW = NC * NS

B = BATCH * HIST
B_PER_W = B // NW
CHUNK = 128
NCHUNK = B_PER_W // CHUNK


def _make_kernel():
    mesh = plsc.VectorSubcoreMesh(core_axis_name="c", subcore_axis_name="s")

    @functools.partial(
        pl.kernel,
        mesh=mesh,
        out_type=jax.ShapeDtypeStruct((B, EMBED_DIM), jnp.float32),
        scratch_types=[
            pltpu.VMEM((NCHUNK, CHUNK), jnp.int32),
            pltpu.VMEM((CHUNK, EMBED_DIM), jnp.float32),
            pltpu.SemaphoreType.DMA,
        ],
        compiler_params=pltpu.CompilerParams(use_tc_tiling_on_sc=False),
    )
    def emb_gather(idx_hbm, table_hbm, out_hbm, idx_v, rows_v, sem):
        wid = lax.axis_index("s") * NC + lax.axis_index("c")
        base = wid * B_PER_W
        pltpu.sync_copy(idx_hbm.at[wid], idx_v)

        def chunk_body(j, carry):
            pltpu.async_copy(table_hbm.at[idx_v.at[j]], rows_v, sem).wait()
            pltpu.sync_copy(rows_v, out_hbm.at[pl.ds(base + j * CHUNK, CHUNK)])
            return carry

        lax.fori_loop(0, NCHUNK, chunk_body, 0)

    return emb_gather


_emb_gather = _make_kernel()


@jax.jit
def kernel(token_ids, lookup_table):
    idx3 = token_ids.reshape(NW, NCHUNK, CHUNK)
    out = _emb_gather(idx3, lookup_table)
    return out.reshape(BATCH, HIST, EMBED_DIM)


# speedup vs baseline: 1.8706x; 1.1119x over previous
import functools

import jax
import jax.numpy as jnp
from jax import lax
from jax.experimental import pallas as pl
from jax.experimental.pallas import tpu as pltpu
from jax.experimental.pallas import tpu_sc as plsc

NUM_EMBEDDINGS = 1000000
EMBED_DIM = 64
BATCH = 16384
HIST = 50

NC = 2
NS = 16
NW = NC * NS

B = BATCH * HIST
B_PER_W = B // NW
CHUNK = 128
NCHUNK = B_PER_W // CHUNK


NBUF = 4
NGROUP = NCHUNK // NBUF


def _make_kernel():
    mesh = plsc.VectorSubcoreMesh(core_axis_name="c", subcore_axis_name="s")

    @functools.partial(
        pl.kernel,
        mesh=mesh,
        out_type=jax.ShapeDtypeStruct((B, EMBED_DIM), jnp.float32),
        scratch_types=[
            pltpu.VMEM((NCHUNK, CHUNK), jnp.int32),
            [pltpu.VMEM((CHUNK, EMBED_DIM), jnp.float32) for _ in range(NBUF)],
            [pltpu.SemaphoreType.DMA for _ in range(NBUF)],
            [pltpu.SemaphoreType.DMA for _ in range(NBUF)],
        ],
        compiler_params=pltpu.CompilerParams(use_tc_tiling_on_sc=False),
    )
    def emb_gather(idx_hbm, table_hbm, out_hbm, idx_v, rows, sem_g, sem_w):
        wid = lax.axis_index("s") * NC + lax.axis_index("c")
        base = wid * B_PER_W
        pltpu.sync_copy(idx_hbm.at[wid], idx_v)

        def fire_gather(j, b):
            pltpu.async_copy(table_hbm.at[idx_v.at[j]], rows[b], sem_g[b])

        def wait_gather(b):
            pltpu.make_async_copy(table_hbm.at[idx_v.at[0]], rows[b], sem_g[b]).wait()

        def fire_writeback(j, b):
            pltpu.async_copy(rows[b], out_hbm.at[pl.ds(base + j * CHUNK, CHUNK)], sem_w[b])

        def wait_writeback(b):
            pltpu.make_async_copy(
                rows[b], out_hbm.at[pl.ds(base, CHUNK)], sem_w[b]
            ).wait()

        for b in range(NBUF):
            fire_gather(b, b)

        def outer(g, carry):
            j0 = g * NBUF
            for b in range(NBUF):
                wait_gather(b)
                fire_writeback(j0 + b, b)
            for b in range(NBUF):
                wait_writeback(b)
                fire_gather(j0 + NBUF + b, b)
            return carry

        lax.fori_loop(0, NGROUP - 1, outer, 0)

        j0 = (NGROUP - 1) * NBUF
        for b in range(NBUF):
            wait_gather(b)
            fire_writeback(j0 + b, b)
        for b in range(NBUF):
            wait_writeback(b)

    return emb_gather


_emb_gather = _make_kernel()


@jax.jit
def kernel(token_ids, lookup_table):
    idx3 = token_ids.reshape(NW, NCHUNK, CHUNK)
    out = _emb_gather(idx3, lookup_table)
    return out.reshape(BATCH, HIST, EMBED_DIM)
